# baseline (speedup 1.0000x reference)
import functools

import jax
import jax.numpy as jnp
from jax import lax
from jax.experimental import pallas as pl
from jax.experimental.pallas import tpu as pltpu
from jax.experimental.pallas import tpu_sc as plsc

HIST_ROWS = 10
NUM_WORKERS = 32
LANES = 16
IDX_CHUNK = 128

TC_ROWS = 8192
TC_BLOCK = 512


def _build_sc_kernel(n_rows, d, row0, sc_rows):
    rows_per_w = sc_rows // NUM_WORKERS
    n_chunks = rows_per_w // IDX_CHUNK
    mesh = plsc.VectorSubcoreMesh(core_axis_name="c", subcore_axis_name="s", num_cores=2)

    @functools.partial(
        pl.kernel,
        mesh=mesh,
        out_type=jax.ShapeDtypeStruct((sc_rows, d), jnp.float32),
        scratch_types=[
            pltpu.VMEM((rows_per_w, d), jnp.float32),
            pltpu.VMEM((HIST_ROWS, n_chunks, IDX_CHUNK), jnp.int32),
            pltpu.SemaphoreType.DMA,
        ],
    )
    def sc_sum(ap_hbm, hist_hbm, out_hbm, acc, idx, sem):
        wid = lax.axis_index("s") * 2 + lax.axis_index("c")
        base = wid * rows_per_w

        iota = lax.iota(jnp.int32, LANES)
        for r in range(HIST_ROWS):
            for j in range(n_chunks):
                for l in range(IDX_CHUNK // LANES):
                    off = r * n_rows + row0 + j * IDX_CHUNK + l * LANES
                    idx[r, j, pl.ds(l * LANES, LANES)] = iota + (base + off)

        pltpu.sync_copy(ap_hbm.at[pl.ds(row0 + base, rows_per_w)], acc)

        copies = []
        for r in range(HIST_ROWS):
            for j in range(n_chunks):
                copies.append(
                    pltpu.async_copy(
                        hist_hbm.at[idx.at[r, j]],
                        acc.at[pl.ds(j * IDX_CHUNK, IDX_CHUNK)],
                        sem,
                        add=True,
                    )
                )
        for cp in copies:
            cp.wait()

        pltpu.sync_copy(acc, out_hbm.at[pl.ds(base, rows_per_w)])

    return sc_sum


def _tc_body(ap_ref, hist_ref, out_ref):
    out_ref[...] = ap_ref[...] + jnp.sum(hist_ref[...], axis=0)


def _tc_sum(ap, hist, d):
    grid = (TC_ROWS // TC_BLOCK,)
    return pl.pallas_call(
        _tc_body,
        grid=grid,
        in_specs=[
            pl.BlockSpec((TC_BLOCK, d), lambda i: (i, 0)),
            pl.BlockSpec((HIST_ROWS, TC_BLOCK, d), lambda i: (0, i, 0)),
        ],
        out_specs=pl.BlockSpec((TC_BLOCK, d), lambda i: (i, 0)),
        out_shape=jax.ShapeDtypeStruct((TC_ROWS, d), jnp.float32),
    )(ap, hist)


@jax.jit
def kernel(action_potential, action_potential_history):
    n_rows, d = action_potential.shape
    sc_rows = n_rows - TC_ROWS
    hist2d = action_potential_history.reshape(-1, d)
    sc_out = _build_sc_kernel(n_rows, d, TC_ROWS, sc_rows)(
        action_potential, hist2d
    )
    tc_out = _tc_sum(action_potential, action_potential_history, d)
    return jnp.concatenate([tc_out, sc_out], axis=0)


# speedup vs baseline: 1.5300x; 1.5300x over previous
import functools

import jax
import jax.numpy as jnp
from jax import lax
from jax.experimental import pallas as pl
from jax.experimental.pallas import tpu as pltpu
from jax.experimental.pallas import tpu_sc as plsc

HIST_ROWS = 10
NUM_WORKERS = 32
LANES = 16
IDX_CHUNK = 128

TC_ROWS = 16384
TC_BLOCK = 512


def _build_sc_kernel(n_rows, d, row0, sc_rows):
    rows_per_w = sc_rows // NUM_WORKERS
    n_chunks = rows_per_w // IDX_CHUNK
    mesh = plsc.VectorSubcoreMesh(core_axis_name="c", subcore_axis_name="s", num_cores=2)

    @functools.partial(
        pl.kernel,
        mesh=mesh,
        out_type=jax.ShapeDtypeStruct((sc_rows, d), jnp.float32),
        scratch_types=[
            pltpu.VMEM((rows_per_w, d), jnp.float32),
            pltpu.VMEM((HIST_ROWS, n_chunks, IDX_CHUNK), jnp.int32),
            pltpu.SemaphoreType.DMA,
        ],
    )
    def sc_sum(ap_hbm, hist_hbm, out_hbm, acc, idx, sem):
        wid = lax.axis_index("s") * 2 + lax.axis_index("c")
        base = wid * rows_per_w

        iota = lax.iota(jnp.int32, LANES)
        for r in range(HIST_ROWS):
            for j in range(n_chunks):
                for l in range(IDX_CHUNK // LANES):
                    off = r * n_rows + row0 + j * IDX_CHUNK + l * LANES
                    idx[r, j, pl.ds(l * LANES, LANES)] = iota + (base + off)

        pltpu.sync_copy(ap_hbm.at[pl.ds(row0 + base, rows_per_w)], acc)

        copies = []
        for r in range(HIST_ROWS):
            for j in range(n_chunks):
                copies.append(
                    pltpu.async_copy(
                        hist_hbm.at[idx.at[r, j]],
                        acc.at[pl.ds(j * IDX_CHUNK, IDX_CHUNK)],
                        sem,
                        add=True,
                    )
                )
        for cp in copies:
            cp.wait()

        pltpu.sync_copy(acc, out_hbm.at[pl.ds(base, rows_per_w)])

    return sc_sum


def _tc_body(ap_ref, hist_ref, out_ref):
    out_ref[...] = ap_ref[...] + jnp.sum(hist_ref[...], axis=0)


def _tc_sum(ap, hist, d):
    grid = (TC_ROWS // TC_BLOCK,)
    return pl.pallas_call(
        _tc_body,
        grid=grid,
        in_specs=[
            pl.BlockSpec((TC_BLOCK, d), lambda i: (i, 0)),
            pl.BlockSpec((HIST_ROWS, TC_BLOCK, d), lambda i: (0, i, 0)),
        ],
        out_specs=pl.BlockSpec((TC_BLOCK, d), lambda i: (i, 0)),
        out_shape=jax.ShapeDtypeStruct((TC_ROWS, d), jnp.float32),
    )(ap, hist)


@jax.jit
def kernel(action_potential, action_potential_history):
    n_rows, d = action_potential.shape
    sc_rows = n_rows - TC_ROWS
    hist2d = action_potential_history.reshape(-1, d)
    tc_out = _tc_sum(action_potential, action_potential_history, d)
    return tc_out


# speedup vs baseline: 1.7936x; 1.1723x over previous
import functools

import jax
import jax.numpy as jnp
from jax import lax
from jax.experimental import pallas as pl
from jax.experimental.pallas import tpu as pltpu
from jax.experimental.pallas import tpu_sc as plsc

HIST_ROWS = 10
NUM_WORKERS = 32
LANES = 16
IDX_CHUNK = 128

TC_ROWS = 16384
TC_BLOCK = 1024


def _build_sc_kernel(n_rows, d, row0, sc_rows):
    rows_per_w = sc_rows // NUM_WORKERS
    n_chunks = rows_per_w // IDX_CHUNK
    mesh = plsc.VectorSubcoreMesh(core_axis_name="c", subcore_axis_name="s", num_cores=2)

    @functools.partial(
        pl.kernel,
        mesh=mesh,
        out_type=jax.ShapeDtypeStruct((sc_rows, d), jnp.float32),
        scratch_types=[
            pltpu.VMEM((rows_per_w, d), jnp.float32),
            pltpu.VMEM((HIST_ROWS, n_chunks, IDX_CHUNK), jnp.int32),
            pltpu.SemaphoreType.DMA,
        ],
    )
    def sc_sum(ap_hbm, hist_hbm, out_hbm, acc, idx, sem):
        wid = lax.axis_index("s") * 2 + lax.axis_index("c")
        base = wid * rows_per_w

        iota = lax.iota(jnp.int32, LANES)
        for r in range(HIST_ROWS):
            for j in range(n_chunks):
                for l in range(IDX_CHUNK // LANES):
                    off = r * n_rows + row0 + j * IDX_CHUNK + l * LANES
                    idx[r, j, pl.ds(l * LANES, LANES)] = iota + (base + off)

        pltpu.sync_copy(ap_hbm.at[pl.ds(row0 + base, rows_per_w)], acc)

        copies = []
        for r in range(HIST_ROWS):
            for j in range(n_chunks):
                copies.append(
                    pltpu.async_copy(
                        hist_hbm.at[idx.at[r, j]],
                        acc.at[pl.ds(j * IDX_CHUNK, IDX_CHUNK)],
                        sem,
                        add=True,
                    )
                )
        for cp in copies:
            cp.wait()

        pltpu.sync_copy(acc, out_hbm.at[pl.ds(base, rows_per_w)])

    return sc_sum


def _tc_body(ap_ref, hist_ref, out_ref):
    out_ref[...] = ap_ref[...] + jnp.sum(hist_ref[...], axis=0)


def _tc_sum(ap, hist, d):
    grid = (TC_ROWS // TC_BLOCK,)
    return pl.pallas_call(
        _tc_body,
        grid=grid,
        in_specs=[
            pl.BlockSpec((TC_BLOCK, d), lambda i: (i, 0)),
            pl.BlockSpec((HIST_ROWS, TC_BLOCK, d), lambda i: (0, i, 0)),
        ],
        out_specs=pl.BlockSpec((TC_BLOCK, d), lambda i: (i, 0)),
        out_shape=jax.ShapeDtypeStruct((TC_ROWS, d), jnp.float32),
    )(ap, hist)


@jax.jit
def kernel(action_potential, action_potential_history):
    n_rows, d = action_potential.shape
    sc_rows = n_rows - TC_ROWS
    hist2d = action_potential_history.reshape(-1, d)
    tc_out = _tc_sum(action_potential, action_potential_history, d)
    return tc_out


# speedup vs baseline: 1.8495x; 1.0312x over previous
import functools

import jax
import jax.numpy as jnp
from jax import lax
from jax.experimental import pallas as pl
from jax.experimental.pallas import tpu as pltpu
from jax.experimental.pallas import tpu_sc as plsc

HIST_ROWS = 10
NUM_WORKERS = 32
LANES = 16
IDX_CHUNK = 128

TC_ROWS = 16384
TC_BLOCK = 2048


def _build_sc_kernel(n_rows, d, row0, sc_rows):
    rows_per_w = sc_rows // NUM_WORKERS
    n_chunks = rows_per_w // IDX_CHUNK
    mesh = plsc.VectorSubcoreMesh(core_axis_name="c", subcore_axis_name="s", num_cores=2)

    @functools.partial(
        pl.kernel,
        mesh=mesh,
        out_type=jax.ShapeDtypeStruct((sc_rows, d), jnp.float32),
        scratch_types=[
            pltpu.VMEM((rows_per_w, d), jnp.float32),
            pltpu.VMEM((HIST_ROWS, n_chunks, IDX_CHUNK), jnp.int32),
            pltpu.SemaphoreType.DMA,
        ],
    )
    def sc_sum(ap_hbm, hist_hbm, out_hbm, acc, idx, sem):
        wid = lax.axis_index("s") * 2 + lax.axis_index("c")
        base = wid * rows_per_w

        iota = lax.iota(jnp.int32, LANES)
        for r in range(HIST_ROWS):
            for j in range(n_chunks):
                for l in range(IDX_CHUNK // LANES):
                    off = r * n_rows + row0 + j * IDX_CHUNK + l * LANES
                    idx[r, j, pl.ds(l * LANES, LANES)] = iota + (base + off)

        pltpu.sync_copy(ap_hbm.at[pl.ds(row0 + base, rows_per_w)], acc)

        copies = []
        for r in range(HIST_ROWS):
            for j in range(n_chunks):
                copies.append(
                    pltpu.async_copy(
                        hist_hbm.at[idx.at[r, j]],
                        acc.at[pl.ds(j * IDX_CHUNK, IDX_CHUNK)],
                        sem,
                        add=True,
                    )
                )
        for cp in copies:
            cp.wait()

        pltpu.sync_copy(acc, out_hbm.at[pl.ds(base, rows_per_w)])

    return sc_sum


def _tc_body(ap_ref, hist_ref, out_ref):
    out_ref[...] = ap_ref[...] + jnp.sum(hist_ref[...], axis=0)


def _tc_sum(ap, hist, d):
    grid = (TC_ROWS // TC_BLOCK,)
    return pl.pallas_call(
        _tc_body,
        grid=grid,
        in_specs=[
            pl.BlockSpec((TC_BLOCK, d), lambda i: (i, 0)),
            pl.BlockSpec((HIST_ROWS, TC_BLOCK, d), lambda i: (0, i, 0)),
        ],
        out_specs=pl.BlockSpec((TC_BLOCK, d), lambda i: (i, 0)),
        out_shape=jax.ShapeDtypeStruct((TC_ROWS, d), jnp.float32),
    )(ap, hist)


@jax.jit
def kernel(action_potential, action_potential_history):
    n_rows, d = action_potential.shape
    sc_rows = n_rows - TC_ROWS
    hist2d = action_potential_history.reshape(-1, d)
    tc_out = _tc_sum(action_potential, action_potential_history, d)
    return tc_out
